# baseline (speedup 1.0000x reference)
import functools

import jax
import jax.numpy as jnp
from jax import lax
from jax.experimental import pallas as pl
from jax.experimental.pallas import tpu as pltpu
from jax.experimental.pallas import tpu_sc as plsc

_NUM_WORKERS = 32
_LANES = 16


def _bag_body(l_half, rows_per_worker, inp_hbm, tbl_hbm, out_hbm,
              idx_v, rows_v, acc_v, sem):
  c = lax.axis_index("c")
  s = lax.axis_index("s")
  wid = s * 2 + c
  base = wid * rows_per_worker
  embed = tbl_hbm.shape[1]
  n_vregs = embed // _LANES

  def row_body(r, carry):
    pltpu.sync_copy(inp_hbm.at[base + r], idx_v)
    cp0 = pltpu.async_copy(tbl_hbm.at[idx_v.at[0]],
                           rows_v.at[pl.ds(0, l_half)], sem)
    cp1 = pltpu.async_copy(tbl_hbm.at[idx_v.at[1]],
                           rows_v.at[pl.ds(l_half, l_half)], sem)
    cp0.wait()
    cp1.wait()

    def red(i, accs):
      cur = list(accs)
      for u in range(4):
        row = i * 4 + u
        for j in range(n_vregs):
          cur[j] = cur[j] + rows_v[row, pl.ds(_LANES * j, _LANES)]
      return tuple(cur)

    zeros = tuple(jnp.zeros((_LANES,), jnp.float32) for _ in range(n_vregs))
    accs = lax.fori_loop(0, (2 * l_half) // 4, red, zeros)
    for j in range(n_vregs):
      acc_v[r, pl.ds(_LANES * j, _LANES)] = accs[j]
    return carry

  lax.fori_loop(0, rows_per_worker, row_body, 0)
  pltpu.sync_copy(acc_v, out_hbm.at[pl.ds(base, rows_per_worker)])


def _bag(idx, emb_table):
  b, _, l_half = idx.shape
  embed = emb_table.shape[1]
  rows_per_worker = b // _NUM_WORKERS
  mesh = plsc.VectorSubcoreMesh(core_axis_name="c", subcore_axis_name="s")
  return pl.kernel(
      functools.partial(_bag_body, l_half, rows_per_worker),
      out_type=jax.ShapeDtypeStruct((b, embed), jnp.float32),
      mesh=mesh,
      compiler_params=pltpu.CompilerParams(use_tc_tiling_on_sc=False),
      scratch_types=[
          pltpu.VMEM((2, l_half), jnp.int32),
          pltpu.VMEM((2 * l_half, embed), jnp.float32),
          pltpu.VMEM((rows_per_worker, embed), jnp.float32),
          pltpu.SemaphoreType.DMA,
      ],
  )(idx, emb_table)


def _proj_body(x_ref, w_ref, b_ref, o_ref):
  o_ref[...] = lax.dot_general(
      x_ref[...].astype(jnp.bfloat16), w_ref[...].astype(jnp.bfloat16),
      (((1,), (1,)), ((), ())),
      preferred_element_type=jnp.float32) + b_ref[...]


def _proj(x, w, bias):
  b, embed = x.shape
  v = w.shape[0]
  nb = 4096
  return pl.pallas_call(
      _proj_body,
      grid=(pl.cdiv(v, nb),),
      in_specs=[
          pl.BlockSpec((b, embed), lambda i: (0, 0)),
          pl.BlockSpec((nb, embed), lambda i: (i, 0)),
          pl.BlockSpec((1, nb), lambda i: (0, i)),
      ],
      out_specs=pl.BlockSpec((b, nb), lambda i: (0, i)),
      out_shape=jax.ShapeDtypeStruct((b, v), jnp.float32),
      compiler_params=pltpu.CompilerParams(
          dimension_semantics=("arbitrary",)),
  )(x, w, bias)


def kernel(inputs, emb_table, W, b):
  bsz, l = inputs.shape
  idx = inputs.astype(jnp.int32).reshape(bsz, 2, l // 2)
  bag = _bag(idx, emb_table)
  return _proj(bag, W, b.reshape(1, -1))


# speedup vs baseline: 1.0396x; 1.0396x over previous
import functools

import jax
import jax.numpy as jnp
from jax import lax
from jax.experimental import pallas as pl
from jax.experimental.pallas import tpu as pltpu
from jax.experimental.pallas import tpu_sc as plsc

_NUM_WORKERS = 32
_LANES = 16


def _bag_body(seq_len, rows_per_worker, inp_hbm, tbl_hbm, out_hbm,
              idx_v, rows_v, acc_v, sem0, sem1):
  c = lax.axis_index("c")
  s = lax.axis_index("s")
  wid = s * 2 + c
  base = wid * rows_per_worker
  embed = tbl_hbm.shape[1]
  n_vregs = embed // _LANES
  l0 = min(seq_len, 128)
  l1 = seq_len - l0
  sems = (sem0, sem1)

  pltpu.sync_copy(inp_hbm.at[pl.ds(base, rows_per_worker)], idx_v)

  def fire(r, buf):
    pltpu.async_copy(tbl_hbm.at[idx_v.at[r, pl.ds(0, l0)]],
                     rows_v.at[buf, pl.ds(0, l0)], sems[buf])
    pltpu.async_copy(tbl_hbm.at[idx_v.at[r, pl.ds(l0, l1)]],
                     rows_v.at[buf, pl.ds(l0, l1)], sems[buf])

  def drain(buf):
    pltpu.make_async_copy(tbl_hbm.at[pl.ds(0, seq_len)],
                          rows_v.at[buf], sems[buf]).wait()

  def reduce_row(r, buf):
    def red(i, accs):
      cur = list(accs)
      for u in range(4):
        row = i * 4 + u
        for j in range(n_vregs):
          cur[j] = cur[j] + rows_v[buf, row, pl.ds(_LANES * j, _LANES)]
      return tuple(cur)

    zeros = tuple(jnp.zeros((_LANES,), jnp.float32) for _ in range(n_vregs))
    accs = lax.fori_loop(0, seq_len // 4, red, zeros)
    for j in range(n_vregs):
      acc_v[r, pl.ds(_LANES * j, _LANES)] = accs[j]

  fire(0, 0)

  def pair_body(p, carry):
    r0 = 2 * p
    for buf in range(2):
      r = r0 + buf
      drain(buf)

      @pl.when(r + 1 < rows_per_worker)
      def _prefetch():
        fire(r + 1, 1 - buf)

      reduce_row(r, buf)
    return carry

  lax.fori_loop(0, rows_per_worker // 2, pair_body, 0)
  pltpu.sync_copy(acc_v, out_hbm.at[pl.ds(base, rows_per_worker)])


def _bag(idx, emb_table):
  b, seq_len = idx.shape
  embed = emb_table.shape[1]
  rows_per_worker = b // _NUM_WORKERS
  mesh = plsc.VectorSubcoreMesh(core_axis_name="c", subcore_axis_name="s")
  return pl.kernel(
      functools.partial(_bag_body, seq_len, rows_per_worker),
      out_type=jax.ShapeDtypeStruct((b, embed), jnp.float32),
      mesh=mesh,
      compiler_params=pltpu.CompilerParams(use_tc_tiling_on_sc=False),
      scratch_types=[
          pltpu.VMEM((rows_per_worker, seq_len), jnp.int32),
          pltpu.VMEM((2, seq_len, embed), jnp.float32),
          pltpu.VMEM((rows_per_worker, embed), jnp.float32),
          pltpu.SemaphoreType.DMA,
          pltpu.SemaphoreType.DMA,
      ],
  )(idx, emb_table)


def _proj_body(x_ref, w_ref, b_ref, o_ref):
  o_ref[...] = lax.dot_general(
      x_ref[...].astype(jnp.bfloat16), w_ref[...].astype(jnp.bfloat16),
      (((1,), (1,)), ((), ())),
      preferred_element_type=jnp.float32) + b_ref[...]


def _proj(x, w, bias):
  b, embed = x.shape
  v = w.shape[0]
  nb = 4096
  return pl.pallas_call(
      _proj_body,
      grid=(pl.cdiv(v, nb),),
      in_specs=[
          pl.BlockSpec((b, embed), lambda i: (0, 0)),
          pl.BlockSpec((nb, embed), lambda i: (i, 0)),
          pl.BlockSpec((1, nb), lambda i: (0, i)),
      ],
      out_specs=pl.BlockSpec((b, nb), lambda i: (0, i)),
      out_shape=jax.ShapeDtypeStruct((b, v), jnp.float32),
      compiler_params=pltpu.CompilerParams(
          dimension_semantics=("arbitrary",)),
  )(x, w, bias)


def kernel(inputs, emb_table, W, b):
  bag = _bag(inputs.astype(jnp.int32), emb_table)
  return _proj(bag, W, b.reshape(1, -1))


# speedup vs baseline: 2.5598x; 2.4622x over previous
import functools

import jax
import jax.numpy as jnp
from jax import lax
from jax.experimental import pallas as pl
from jax.experimental.pallas import tpu as pltpu
from jax.experimental.pallas import tpu_sc as plsc

_NUM_WORKERS = 32
_LANES = 16


def _bag_body(seq_len, l_pad, rows_per_worker, inp_hbm, tbl_hbm, out_hbm,
              idx_v, rows_v, acc_v, sem0, sem1):
  c = lax.axis_index("c")
  s = lax.axis_index("s")
  wid = s * 2 + c
  base = wid * rows_per_worker
  embed = tbl_hbm.shape[1]
  n_vregs = embed // _LANES
  l0 = min(seq_len, 128)
  l1 = seq_len - l0
  sems = (sem0, sem1)

  pltpu.sync_copy(inp_hbm.at[pl.ds(base, rows_per_worker)], idx_v)

  def fire(r, buf):
    pltpu.async_copy(tbl_hbm.at[idx_v.at[r, pl.ds(0, l0)]],
                     rows_v.at[buf, pl.ds(0, l0)], sems[buf])
    pltpu.async_copy(tbl_hbm.at[idx_v.at[r, pl.ds(l0, l1)]],
                     rows_v.at[buf, pl.ds(l0, l1)], sems[buf])

  def drain(buf):
    pltpu.make_async_copy(tbl_hbm.at[pl.ds(0, seq_len)],
                          rows_v.at[buf], sems[buf]).wait()

  def reduce_row(r, buf):
    def red(i, accs):
      cur = list(accs)
      for u in range(8):
        row = i * 8 + u
        for j in range(n_vregs):
          cur[j] = cur[j] + rows_v[buf, row, pl.ds(_LANES * j, _LANES)]
      return tuple(cur)

    zeros = tuple(jnp.zeros((_LANES,), jnp.float32) for _ in range(n_vregs))
    accs = lax.fori_loop(0, seq_len // 8, red, zeros)
    p = r // 2
    for j in range(n_vregs):
      acc_v[p, pl.ds(buf * embed + _LANES * j, _LANES)] = accs[j]

  fire(0, 0)

  def pair_body(p, carry):
    r0 = 2 * p
    for buf in range(2):
      r = r0 + buf
      drain(buf)

      @pl.when(r + 1 < rows_per_worker)
      def _prefetch():
        fire(r + 1, 1 - buf)

      reduce_row(r, buf)
    return carry

  lax.fori_loop(0, rows_per_worker // 2, pair_body, 0)
  pltpu.sync_copy(acc_v, out_hbm.at[pl.ds(base // 2, rows_per_worker // 2)])


def _bag(idx, emb_table, seq_len):
  b, l_pad = idx.shape
  embed = emb_table.shape[1]
  rows_per_worker = b // _NUM_WORKERS
  mesh = plsc.VectorSubcoreMesh(core_axis_name="c", subcore_axis_name="s")
  return pl.kernel(
      functools.partial(_bag_body, seq_len, l_pad, rows_per_worker),
      out_type=jax.ShapeDtypeStruct((b // 2, 2 * embed), jnp.float32),
      mesh=mesh,
      compiler_params=pltpu.CompilerParams(use_tc_tiling_on_sc=False),
      scratch_types=[
          pltpu.VMEM((rows_per_worker, l_pad), jnp.int32),
          pltpu.VMEM((2, seq_len, embed), jnp.float32),
          pltpu.VMEM((rows_per_worker // 2, 2 * embed), jnp.float32),
          pltpu.SemaphoreType.DMA,
          pltpu.SemaphoreType.DMA,
      ],
  )(idx, emb_table)


def _proj_body(x_ref, wt_ref, b_ref, o_ref):
  mm = lax.dot_general(
      wt_ref[...].astype(jnp.bfloat16), x_ref[...].astype(jnp.bfloat16),
      (((0,), (1,)), ((), ())),
      preferred_element_type=jnp.float32)
  ones = jnp.ones((1, x_ref.shape[0]), jnp.bfloat16)
  bias = lax.dot_general(
      b_ref[...].astype(jnp.bfloat16), ones,
      (((0,), (0,)), ((), ())),
      preferred_element_type=jnp.float32)
  o_ref[...] = mm + bias


def _proj(x, wt, bias):
  b, embed = x.shape
  v = wt.shape[1]
  nb = 4096
  return pl.pallas_call(
      _proj_body,
      grid=(pl.cdiv(v, nb),),
      in_specs=[
          pl.BlockSpec((b, embed), lambda i: (0, 0)),
          pl.BlockSpec((embed, nb), lambda i: (0, i)),
          pl.BlockSpec((1, nb), lambda i: (0, i)),
      ],
      out_specs=pl.BlockSpec((nb, b), lambda i: (i, 0)),
      out_shape=jax.ShapeDtypeStruct((v, b), jnp.float32),
      compiler_params=pltpu.CompilerParams(
          dimension_semantics=("arbitrary",)),
  )(x, wt, bias)


def kernel(inputs, emb_table, W, b):
  bsz, seq_len = inputs.shape
  pad = (-seq_len) % 128
  idx = jnp.pad(inputs.astype(jnp.int32), ((0, 0), (0, pad)))
  bag = _bag(idx, emb_table, seq_len).reshape(bsz, -1)
  return _proj(bag, W.T, b.reshape(1, -1)).T
